# Initial kernel scaffold; baseline (speedup 1.0000x reference)
import jax
import jax.numpy as jnp
from jax.experimental import pallas as pl


def kernel(X, A, kernel):
    raise NotImplementedError("write your pallas kernel here")


# speedup vs baseline: 2.3285x; 2.3285x over previous
import functools

import jax
import jax.numpy as jnp
from jax import lax
from jax.experimental import pallas as pl
from jax.experimental.pallas import tpu as pltpu
from jax.experimental.pallas import tpu_sc as plsc

N = 10000
D = 128
K = 5000
NPAD = 10240
NV = NPAD // 16
NC = 2
NW = 32
FCHUNKS = (K + 7) // 8
ACHUNKS = K // 2
CCHUNKS = (K + 15) // 16


def _score_body(x_ref, p_ref, keys_ref, feat_ref):
    x = x_ref[...]
    pn = p_ref[...]
    y = jnp.dot(x, pn, preferred_element_type=jnp.float32)
    feat_ref[...] = x * jnp.tanh(y)
    u = lax.bitcast_convert_type(y, jnp.uint32)
    key = jnp.where(u >= jnp.uint32(0x80000000), ~u, u | jnp.uint32(0x80000000))
    keys_ref[...] = jnp.concatenate(
        [key, jnp.zeros((NPAD - N, 1), jnp.uint32)], axis=0)


_score = pl.pallas_call(
    _score_body,
    out_shape=(
        jax.ShapeDtypeStruct((NPAD, 1), jnp.uint32),
        jax.ShapeDtypeStruct((N, D), jnp.float32),
    ),
)


def _radix_pass(keys_v, hist_v, shift, nbits, cand_shift, cand_prefix, kk,
                first):
    nb = 1 << nbits
    nb16 = nb // 16
    zeros = jnp.zeros((16,), jnp.int32)
    ones = jnp.ones((16,), jnp.int32)
    fmask = jnp.uint32(nb - 1)

    def zero_body(i, _):
        hist_v[pl.ds(i * 16, 16)] = zeros
        return 0

    lax.fori_loop(0, nb16, zero_body, 0)

    def build_body(i, _):
        kv = keys_v[pl.ds(i * 16, 16)]
        b = ((kv >> jnp.uint32(shift)) & fmask).astype(jnp.int32)
        if first:
            plsc.addupdate_scatter(hist_v, [b], ones)
        else:
            m = (kv >> jnp.uint32(cand_shift)) == cand_prefix
            plsc.addupdate_scatter(hist_v, [b], ones, mask=m)
        return 0

    lax.fori_loop(0, NV, build_body, 0)

    def scan_body(i, carry):
        run, found, bucket, c_above = carry
        v = nb16 - 1 - i
        h = hist_v[pl.ds(v * 16, 16)]
        hr = lax.rev(h, (0,))
        s = run + plsc.cumsum(hr)
        okm = s >= kk
        anyok = jnp.max(plsc.all_reduce_population_count(okm)) > 0
        j = jnp.max(plsc.all_reduce_ffs(okm))
        ca = jnp.maximum(run, jnp.max(jnp.where(okm, 0, s)))
        newf = jnp.logical_and(anyok, jnp.logical_not(found))
        bucket = jnp.where(newf, v * 16 + (15 - j), bucket)
        c_above = jnp.where(newf, ca, c_above)
        run = jnp.max(s)
        found = jnp.logical_or(found, anyok)
        return run, found, bucket, c_above

    _, _, bucket, c_above = lax.fori_loop(
        0, nb16, scan_body,
        (jnp.int32(0), False, jnp.int32(0), jnp.int32(0)))
    return bucket, kk - c_above


def _sc_body(keys_hbm, feat_hbm, a_hbm, xp_hbm, ap_hbm,
             keys_v, idx_v, hist_v, fbuf_v, rbuf_v, obuf_v, stage_v,
             sem_f, sem_a):
    wid = lax.axis_index("s") * NC + lax.axis_index("c")

    pltpu.sync_copy(keys_hbm, keys_v)

    kk = jnp.int32(K)
    b1, kk = _radix_pass(keys_v, hist_v, 21, 11, 0, 0, kk, True)
    p1 = b1.astype(jnp.uint32)
    b2, kk = _radix_pass(keys_v, hist_v, 10, 11, 21, p1, kk, False)
    p2 = (p1 << 11) | b2.astype(jnp.uint32)
    b3, kk = _radix_pass(keys_v, hist_v, 0, 10, 10, p2, kk, False)
    thresh = (p2 << 10) | b3.astype(jnp.uint32)
    need = kk

    iota = lax.iota(jnp.int32, 16)

    def comp_body(i, carry):
        gt_cnt, tie_cnt = carry
        kv = keys_v[pl.ds(i * 16, 16)]
        gt_m = kv > thresh
        eq_m = kv == thresh
        gt_i = gt_m.astype(jnp.int32)
        eq_i = eq_m.astype(jnp.int32)
        gt_c = plsc.cumsum(gt_i)
        eq_c = plsc.cumsum(eq_i)
        gt_excl = gt_cnt + gt_c - gt_i
        tie_excl = tie_cnt + eq_c - eq_i
        sel = jnp.logical_or(gt_m, jnp.logical_and(eq_m, tie_excl < need))
        pos = gt_excl + jnp.minimum(tie_excl, need)
        plsc.store_scatter(idx_v, [pos], i * 16 + iota, mask=sel)
        return gt_cnt + jnp.max(gt_c), tie_cnt + jnp.max(eq_c)

    lax.fori_loop(0, NV, comp_body, (jnp.int32(0), jnp.int32(0)))

    def f_body(t, _):
        c = wid + NW * t

        @pl.when(c < FCHUNKS)
        def _():
            pltpu.async_copy(
                feat_hbm.at[idx_v.at[pl.ds(8 * c, 8)]], fbuf_v, sem_f).wait()
            pltpu.sync_copy(fbuf_v, xp_hbm.at[pl.ds(8 * c, 8)])

        return 0

    lax.fori_loop(0, (FCHUNKS + NW - 1) // NW, f_body, 0)

    def a_body(t, _):
        c = wid + NW * t

        @pl.when(c < ACHUNKS)
        def _():
            ridx = plsc.load_gather(
                idx_v, [jnp.minimum(2 * c + iota, K - 1)])
            stage_v[pl.ds(0, 16)] = ridx
            pltpu.async_copy(
                a_hbm.at[stage_v.at[pl.ds(0, 2)]], rbuf_v, sem_a).wait()
            for r in range(2):
                rvec = jnp.full((16,), r, jnp.int32)

                def col_body(j, _, rvec=rvec, r=r):
                    off = jnp.minimum(j * 16, K - 16)
                    col = idx_v[pl.ds(off, 16)]
                    obuf_v[r, pl.ds(off, 16)] = plsc.load_gather(
                        rbuf_v, [rvec, col])
                    return 0

                lax.fori_loop(0, CCHUNKS, col_body, 0)
            pltpu.sync_copy(obuf_v, ap_hbm.at[pl.ds(2 * c, 2)])

        return 0

    lax.fori_loop(0, (ACHUNKS + NW - 1) // NW, a_body, 0)


_pool = pl.kernel(
    _sc_body,
    out_type=(
        jax.ShapeDtypeStruct((K, D), jnp.float32),
        jax.ShapeDtypeStruct((K, K), jnp.float32),
    ),
    mesh=plsc.VectorSubcoreMesh(core_axis_name="c", subcore_axis_name="s"),
    compiler_params=pltpu.CompilerParams(
        needs_layout_passes=False, use_tc_tiling_on_sc=False),
    scratch_types=[
        pltpu.VMEM((NPAD,), jnp.uint32),
        pltpu.VMEM((K,), jnp.int32),
        pltpu.VMEM((2048,), jnp.int32),
        pltpu.VMEM((8, D), jnp.float32),
        pltpu.VMEM((2, N), jnp.float32),
        pltpu.VMEM((2, K), jnp.float32),
        pltpu.VMEM((16,), jnp.int32),
        pltpu.SemaphoreType.DMA,
        pltpu.SemaphoreType.DMA,
    ],
)


@jax.jit
def kernel(X, A, kernel):
    p = kernel / jnp.sqrt(jnp.sum(jnp.square(kernel)) + 1e-12)
    keys2d, feat = _score(X, p)
    xp, ap = _pool(keys2d[:, 0], feat, A)
    return xp, ap


# speedup vs baseline: 5.5620x; 2.3886x over previous
import functools

import jax
import jax.numpy as jnp
from jax import lax
from jax.experimental import pallas as pl
from jax.experimental.pallas import tpu as pltpu
from jax.experimental.pallas import tpu_sc as plsc

N = 10000
D = 128
K = 5000
NPAD = 10240
NV = NPAD // 16
NC = 2
NW = 32
FCHUNKS = (K + 7) // 8
ACHUNKS = K // 2
CCHUNKS = (K + 15) // 16


def _score_body(x_ref, p_ref, keys_ref, feat_ref):
    x = x_ref[...]
    pn = p_ref[...]
    y = jnp.dot(x, pn, preferred_element_type=jnp.float32)
    feat_ref[...] = x * jnp.tanh(y)
    u = lax.bitcast_convert_type(y, jnp.uint32)
    key = jnp.where(u >= jnp.uint32(0x80000000), ~u, u | jnp.uint32(0x80000000))
    keys_ref[...] = jnp.concatenate(
        [key, jnp.zeros((NPAD - N, 1), jnp.uint32)], axis=0)


_score = pl.pallas_call(
    _score_body,
    out_shape=(
        jax.ShapeDtypeStruct((NPAD, 1), jnp.uint32),
        jax.ShapeDtypeStruct((N, D), jnp.float32),
    ),
)


def _radix_pass(keys_v, hist_v, shift, nbits, cand_shift, cand_prefix, kk,
                first):
    nb = 1 << nbits
    nb16 = nb // 16
    zeros = jnp.zeros((16,), jnp.int32)
    ones = jnp.ones((16,), jnp.int32)
    fmask = jnp.uint32(nb - 1)

    def zero_body(i, _):
        hist_v[pl.ds(i * 16, 16)] = zeros
        return 0

    lax.fori_loop(0, nb16, zero_body, 0)

    def build_body(i, _):
        kv = keys_v[pl.ds(i * 16, 16)]
        b = ((kv >> jnp.uint32(shift)) & fmask).astype(jnp.int32)
        if first:
            plsc.addupdate_scatter(hist_v, [b], ones)
        else:
            m = (kv >> jnp.uint32(cand_shift)) == cand_prefix
            plsc.addupdate_scatter(hist_v, [b], ones, mask=m)
        return 0

    lax.fori_loop(0, NV, build_body, 0)

    def scan_body(i, carry):
        run, found, bucket, c_above = carry
        v = nb16 - 1 - i
        h = hist_v[pl.ds(v * 16, 16)]
        hr = lax.rev(h, (0,))
        s = run + plsc.cumsum(hr)
        okm = s >= kk
        anyok = jnp.max(plsc.all_reduce_population_count(okm)) > 0
        j = jnp.max(plsc.all_reduce_ffs(okm))
        ca = jnp.maximum(run, jnp.max(jnp.where(okm, 0, s)))
        newf = jnp.logical_and(anyok, jnp.logical_not(found))
        bucket = jnp.where(newf, v * 16 + (15 - j), bucket)
        c_above = jnp.where(newf, ca, c_above)
        run = jnp.max(s)
        found = jnp.logical_or(found, anyok)
        return run, found, bucket, c_above

    _, _, bucket, c_above = lax.fori_loop(
        0, nb16, scan_body,
        (jnp.int32(0), False, jnp.int32(0), jnp.int32(0)))
    return bucket, kk - c_above


def _sc_body(keys_hbm, feat_hbm, a_hbm, xp_hbm, ap_hbm,
             keys_v, idx_v, hist_v, fbuf_v, rbuf0, rbuf1, wbuf0, wbuf1,
             sem_f, sem_g0, sem_g1, sem_w0, sem_w1):
    wid = lax.axis_index("s") * NC + lax.axis_index("c")

    pltpu.sync_copy(keys_hbm, keys_v)

    kk = jnp.int32(K)
    b1, kk = _radix_pass(keys_v, hist_v, 21, 11, 0, 0, kk, True)
    p1 = b1.astype(jnp.uint32)
    b2, kk = _radix_pass(keys_v, hist_v, 10, 11, 21, p1, kk, False)
    p2 = (p1 << 11) | b2.astype(jnp.uint32)
    b3, kk = _radix_pass(keys_v, hist_v, 0, 10, 10, p2, kk, False)
    thresh = (p2 << 10) | b3.astype(jnp.uint32)
    need = kk

    iota = lax.iota(jnp.int32, 16)

    def comp_body(i, carry):
        gt_cnt, tie_cnt = carry
        kv = keys_v[pl.ds(i * 16, 16)]
        gt_m = kv > thresh
        eq_m = kv == thresh
        gt_i = gt_m.astype(jnp.int32)
        eq_i = eq_m.astype(jnp.int32)
        gt_c = plsc.cumsum(gt_i)
        eq_c = plsc.cumsum(eq_i)
        gt_excl = gt_cnt + gt_c - gt_i
        tie_excl = tie_cnt + eq_c - eq_i
        sel = jnp.logical_or(gt_m, jnp.logical_and(eq_m, tie_excl < need))
        pos = gt_excl + jnp.minimum(tie_excl, need)
        plsc.store_scatter(idx_v, [pos], i * 16 + iota, mask=sel)
        return gt_cnt + jnp.max(gt_c), tie_cnt + jnp.max(eq_c)

    lax.fori_loop(0, NV, comp_body, (jnp.int32(0), jnp.int32(0)))

    def f_body(t, _):
        c = wid + NW * t

        @pl.when(c < FCHUNKS)
        def _():
            pltpu.async_copy(
                feat_hbm.at[idx_v.at[pl.ds(8 * c, 8)]], fbuf_v, sem_f).wait()
            pltpu.sync_copy(fbuf_v, xp_hbm.at[pl.ds(8 * c, 8)])

        return 0

    lax.fori_loop(0, (FCHUNKS + NW - 1) // NW, f_body, 0)

    NG = 20
    GROUPS = FCHUNKS
    NROWS = 8 * NG

    rbufs = (rbuf0, rbuf1)
    wbufs = (wbuf0, wbuf1)
    sem_g = (sem_g0, sem_g1)
    sem_w = (sem_w0, sem_w1)

    def out_row(t):
        return 8 * jnp.minimum(wid + NW * (t // 8), GROUPS - 1) + t % 8

    def row_src(t):
        row = out_row(t)
        vec = idx_v[pl.ds((row // 16) * 16, 16)]
        v = jnp.max(jnp.where(iota == row % 16, vec, 0))
        return a_hbm.at[v // 8, v % 8]

    def out_dst(t):
        row = out_row(t)
        return ap_hbm.at[row // 8, row % 8]

    for b in range(2):
        pltpu.async_copy(row_src(b), rbufs[b], sem_g[b])

    def a_body(u, _):
        for b in range(2):
            t = 2 * u + b
            pltpu.make_async_copy(a_hbm.at[0, 0], rbufs[b], sem_g[b]).wait()

            @pl.when(u > 0)
            def _(b=b):
                pltpu.make_async_copy(wbufs[b], ap_hbm.at[0, 0],
                                      sem_w[b]).wait()

            def col_body(j, _, b=b):
                off = jnp.minimum(j * 16, K - 16)
                col = idx_v[pl.ds(off, 16)]
                wbufs[b][pl.ds(off, 16)] = plsc.load_gather(rbufs[b], [col])
                return 0

            lax.fori_loop(0, CCHUNKS, col_body, 0)
            pltpu.async_copy(wbufs[b], out_dst(t), sem_w[b])

            @pl.when(u < (NROWS - 2) // 2)
            def _(b=b, t=t):
                pltpu.async_copy(row_src(t + 2), rbufs[b], sem_g[b])

        return 0

    lax.fori_loop(0, NROWS // 2, a_body, 0)
    for b in range(2):
        pltpu.make_async_copy(wbufs[b], ap_hbm.at[0, 0], sem_w[b]).wait()


_pool = pl.kernel(
    _sc_body,
    out_type=(
        jax.ShapeDtypeStruct((K, D), jnp.float32),
        jax.ShapeDtypeStruct((625, 8, K), jnp.float32),
    ),
    mesh=plsc.VectorSubcoreMesh(core_axis_name="c", subcore_axis_name="s"),
    compiler_params=pltpu.CompilerParams(needs_layout_passes=False),
    scratch_types=[
        pltpu.VMEM((NPAD,), jnp.uint32),
        pltpu.VMEM((K,), jnp.int32),
        pltpu.VMEM((2048,), jnp.int32),
        pltpu.VMEM((8, D), jnp.float32),
        pltpu.VMEM((N,), jnp.float32),
        pltpu.VMEM((N,), jnp.float32),
        pltpu.VMEM((K,), jnp.float32),
        pltpu.VMEM((K,), jnp.float32),
        pltpu.SemaphoreType.DMA,
        pltpu.SemaphoreType.DMA,
        pltpu.SemaphoreType.DMA,
        pltpu.SemaphoreType.DMA,
        pltpu.SemaphoreType.DMA,
    ],
)


@jax.jit
def kernel(X, A, kernel):
    p = kernel / jnp.sqrt(jnp.sum(jnp.square(kernel)) + 1e-12)
    keys2d, feat = _score(X, p)
    xp, ap = _pool(keys2d[:, 0], feat, A.reshape(1250, 8, N))
    return xp, ap.reshape(K, K)


# speedup vs baseline: 9.8956x; 1.7791x over previous
import functools

import jax
import jax.numpy as jnp
from jax import lax
from jax.experimental import pallas as pl
from jax.experimental.pallas import tpu as pltpu
from jax.experimental.pallas import tpu_sc as plsc

N = 10000
D = 128
K = 5000
NPAD = 10240
NV = NPAD // 16
NC = 2
NW = 32
FCHUNKS = (K + 7) // 8
ACHUNKS = K // 2
CCHUNKS = (K + 15) // 16


def _score_body(x_ref, p_ref, keys_ref, feat_ref):
    x = x_ref[...]
    pn = p_ref[...]
    y = jnp.dot(x, pn, preferred_element_type=jnp.float32)
    feat_ref[...] = x * jnp.tanh(y)
    u = lax.bitcast_convert_type(y, jnp.uint32)
    key = jnp.where(u >= jnp.uint32(0x80000000), ~u, u | jnp.uint32(0x80000000))
    keys_ref[...] = jnp.concatenate(
        [key, jnp.zeros((NPAD - N, 1), jnp.uint32)], axis=0)


_score = pl.pallas_call(
    _score_body,
    out_shape=(
        jax.ShapeDtypeStruct((NPAD, 1), jnp.uint32),
        jax.ShapeDtypeStruct((N, D), jnp.float32),
    ),
)


def _radix_pass(keys_v, hist_v, shift, nbits, cand_shift, cand_prefix, kk,
                first):
    nb = 1 << nbits
    nb16 = nb // 16
    zeros = jnp.zeros((16,), jnp.int32)
    ones = jnp.ones((16,), jnp.int32)
    fmask = jnp.uint32(nb - 1)

    def zero_body(i, _):
        hist_v[pl.ds(i * 16, 16)] = zeros
        return 0

    lax.fori_loop(0, nb16, zero_body, 0)

    def build_body(i, _):
        kv = keys_v[pl.ds(i * 16, 16)]
        b = ((kv >> jnp.uint32(shift)) & fmask).astype(jnp.int32)
        if first:
            plsc.addupdate_scatter(hist_v, [b], ones)
        else:
            m = (kv >> jnp.uint32(cand_shift)) == cand_prefix
            plsc.addupdate_scatter(hist_v, [b], ones, mask=m)
        return 0

    lax.fori_loop(0, NV, build_body, 0)

    def scan_body(i, carry):
        run, found, bucket, c_above = carry
        v = nb16 - 1 - i
        h = hist_v[pl.ds(v * 16, 16)]
        hr = lax.rev(h, (0,))
        s = run + plsc.cumsum(hr)
        okm = s >= kk
        anyok = jnp.max(plsc.all_reduce_population_count(okm)) > 0
        j = jnp.max(plsc.all_reduce_ffs(okm))
        ca = jnp.maximum(run, jnp.max(jnp.where(okm, 0, s)))
        newf = jnp.logical_and(anyok, jnp.logical_not(found))
        bucket = jnp.where(newf, v * 16 + (15 - j), bucket)
        c_above = jnp.where(newf, ca, c_above)
        run = jnp.max(s)
        found = jnp.logical_or(found, anyok)
        return run, found, bucket, c_above

    _, _, bucket, c_above = lax.fori_loop(
        0, nb16, scan_body,
        (jnp.int32(0), False, jnp.int32(0), jnp.int32(0)))
    return bucket, kk - c_above


def _sc_body(keys_hbm, feat_hbm, a_hbm, xp_hbm, ap_hbm,
             keys_v, idx_v, hist_v, fbuf_v, rbuf0, rbuf1, wbuf0, wbuf1,
             sem_f, sem_g0, sem_g1, sem_w0, sem_w1):
    wid = lax.axis_index("s") * NC + lax.axis_index("c")

    pltpu.sync_copy(keys_hbm, keys_v)

    kk = jnp.int32(K)
    b1, kk = _radix_pass(keys_v, hist_v, 21, 11, 0, 0, kk, True)
    p1 = b1.astype(jnp.uint32)
    b2, kk = _radix_pass(keys_v, hist_v, 10, 11, 21, p1, kk, False)
    p2 = (p1 << 11) | b2.astype(jnp.uint32)
    b3, kk = _radix_pass(keys_v, hist_v, 0, 10, 10, p2, kk, False)
    thresh = (p2 << 10) | b3.astype(jnp.uint32)
    need = kk

    iota = lax.iota(jnp.int32, 16)

    def comp_body(i, carry):
        gt_cnt, tie_cnt = carry
        kv = keys_v[pl.ds(i * 16, 16)]
        gt_m = kv > thresh
        eq_m = kv == thresh
        gt_i = gt_m.astype(jnp.int32)
        eq_i = eq_m.astype(jnp.int32)
        gt_c = plsc.cumsum(gt_i)
        eq_c = plsc.cumsum(eq_i)
        gt_excl = gt_cnt + gt_c - gt_i
        tie_excl = tie_cnt + eq_c - eq_i
        sel = jnp.logical_or(gt_m, jnp.logical_and(eq_m, tie_excl < need))
        pos = gt_excl + jnp.minimum(tie_excl, need)
        plsc.store_scatter(idx_v, [pos], i * 16 + iota, mask=sel)
        return gt_cnt + jnp.max(gt_c), tie_cnt + jnp.max(eq_c)

    lax.fori_loop(0, NV, comp_body, (jnp.int32(0), jnp.int32(0)))

    def f_body(t, _):
        c = wid + NW * t

        @pl.when(c < FCHUNKS)
        def _():
            pltpu.async_copy(
                feat_hbm.at[idx_v.at[pl.ds(8 * c, 8)]], fbuf_v, sem_f).wait()
            pltpu.sync_copy(fbuf_v, xp_hbm.at[pl.ds(8 * c, 8)])

        return 0

    lax.fori_loop(0, (FCHUNKS + NW - 1) // NW, f_body, 0)

    NG = 20
    GROUPS = FCHUNKS
    NROWS = 8 * NG

    rbufs = (rbuf0, rbuf1)
    wbufs = (wbuf0, wbuf1)
    sem_g = (sem_g0, sem_g1)
    sem_w = (sem_w0, sem_w1)

    def out_row(t):
        return 8 * jnp.minimum(wid + NW * (t // 8), GROUPS - 1) + t % 8

    def row_src(t):
        row = out_row(t)
        vec = idx_v[pl.ds((row // 16) * 16, 16)]
        v = jnp.max(jnp.where(iota == row % 16, vec, 0))
        return a_hbm.at[v // 8, v % 8]

    def out_dst(t):
        row = out_row(t)
        return ap_hbm.at[row // 8, row % 8]

    for b in range(2):
        pltpu.async_copy(row_src(b), rbufs[b], sem_g[b])

    def a_body(u, _):
        for b in range(2):
            t = 2 * u + b
            pltpu.make_async_copy(a_hbm.at[0, 0], rbufs[b], sem_g[b]).wait()

            @pl.when(u > 0)
            def _(b=b):
                pltpu.make_async_copy(wbufs[b], ap_hbm.at[0, 0],
                                      sem_w[b]).wait()

            @plsc.parallel_loop(0, CCHUNKS, 1, unroll=8)
            def col_body(j, b=b):
                off = jnp.minimum(j * 16, K - 16)
                col = idx_v[pl.ds(off, 16)]
                wbufs[b][pl.ds(off, 16)] = plsc.load_gather(rbufs[b], [col])
            pltpu.async_copy(wbufs[b], out_dst(t), sem_w[b])

            @pl.when(u < (NROWS - 2) // 2)
            def _(b=b, t=t):
                pltpu.async_copy(row_src(t + 2), rbufs[b], sem_g[b])

        return 0

    lax.fori_loop(0, NROWS // 2, a_body, 0)
    for b in range(2):
        pltpu.make_async_copy(wbufs[b], ap_hbm.at[0, 0], sem_w[b]).wait()


_pool = pl.kernel(
    _sc_body,
    out_type=(
        jax.ShapeDtypeStruct((K, D), jnp.float32),
        jax.ShapeDtypeStruct((625, 8, K), jnp.float32),
    ),
    mesh=plsc.VectorSubcoreMesh(core_axis_name="c", subcore_axis_name="s"),
    compiler_params=pltpu.CompilerParams(needs_layout_passes=False),
    scratch_types=[
        pltpu.VMEM((NPAD,), jnp.uint32),
        pltpu.VMEM((K,), jnp.int32),
        pltpu.VMEM((2048,), jnp.int32),
        pltpu.VMEM((8, D), jnp.float32),
        pltpu.VMEM((N,), jnp.float32),
        pltpu.VMEM((N,), jnp.float32),
        pltpu.VMEM((K,), jnp.float32),
        pltpu.VMEM((K,), jnp.float32),
        pltpu.SemaphoreType.DMA,
        pltpu.SemaphoreType.DMA,
        pltpu.SemaphoreType.DMA,
        pltpu.SemaphoreType.DMA,
        pltpu.SemaphoreType.DMA,
    ],
)


@jax.jit
def kernel(X, A, kernel):
    p = kernel / jnp.sqrt(jnp.sum(jnp.square(kernel)) + 1e-12)
    keys2d, feat = _score(X, p)
    xp, ap = _pool(keys2d[:, 0], feat, A.reshape(1250, 8, N))
    return xp, ap.reshape(K, K)


# speedup vs baseline: 12.1602x; 1.2289x over previous
import functools

import jax
import jax.numpy as jnp
from jax import lax
from jax.experimental import pallas as pl
from jax.experimental.pallas import tpu as pltpu
from jax.experimental.pallas import tpu_sc as plsc

N = 10000
D = 128
K = 5000
NPAD = 10240
NV = NPAD // 16
NC = 2
NW = 32
FCHUNKS = (K + 7) // 8
ACHUNKS = K // 2
CCHUNKS = (K + 15) // 16


def _score_body(x_ref, p_ref, keys_ref, feat_ref):
    x = x_ref[...]
    pn = p_ref[...]
    y = jnp.dot(x, pn, preferred_element_type=jnp.float32)
    feat_ref[...] = x * jnp.tanh(y)
    u = lax.bitcast_convert_type(y, jnp.uint32)
    key = jnp.where(u >= jnp.uint32(0x80000000), ~u, u | jnp.uint32(0x80000000))
    keys_ref[...] = jnp.concatenate(
        [key, jnp.zeros((NPAD - N, 1), jnp.uint32)], axis=0)


_score = pl.pallas_call(
    _score_body,
    out_shape=(
        jax.ShapeDtypeStruct((NPAD, 1), jnp.uint32),
        jax.ShapeDtypeStruct((N, D), jnp.float32),
    ),
)


def _radix_pass(keys_v, hist_v, shift, nbits, cand_shift, cand_prefix, kk,
                first):
    nb = 1 << nbits
    nb16 = nb // 16
    zeros = jnp.zeros((16,), jnp.int32)
    ones = jnp.ones((16,), jnp.int32)
    fmask = jnp.uint32(nb - 1)

    def zero_body(i, _):
        hist_v[pl.ds(i * 16, 16)] = zeros
        return 0

    lax.fori_loop(0, nb16, zero_body, 0)

    def build_body(i, _):
        kv = keys_v[pl.ds(i * 16, 16)]
        b = ((kv >> jnp.uint32(shift)) & fmask).astype(jnp.int32)
        if first:
            plsc.addupdate_scatter(hist_v, [b], ones)
        else:
            m = (kv >> jnp.uint32(cand_shift)) == cand_prefix
            plsc.addupdate_scatter(hist_v, [b], ones, mask=m)
        return 0

    lax.fori_loop(0, NV, build_body, 0)

    def scan_body(i, carry):
        run, found, bucket, c_above = carry
        v = nb16 - 1 - i
        h = hist_v[pl.ds(v * 16, 16)]
        hr = lax.rev(h, (0,))
        s = run + plsc.cumsum(hr)
        okm = s >= kk
        anyok = jnp.max(plsc.all_reduce_population_count(okm)) > 0
        j = jnp.max(plsc.all_reduce_ffs(okm))
        ca = jnp.maximum(run, jnp.max(jnp.where(okm, 0, s)))
        newf = jnp.logical_and(anyok, jnp.logical_not(found))
        bucket = jnp.where(newf, v * 16 + (15 - j), bucket)
        c_above = jnp.where(newf, ca, c_above)
        run = jnp.max(s)
        found = jnp.logical_or(found, anyok)
        return run, found, bucket, c_above

    _, _, bucket, c_above = lax.fori_loop(
        0, nb16, scan_body,
        (jnp.int32(0), False, jnp.int32(0), jnp.int32(0)))
    return bucket, kk - c_above


def _sc_body(keys_hbm, feat_hbm, a_hbm, xp_hbm, ap_hbm,
             keys_v, idx_v, hist_v, fbuf0, fbuf1, rbuf0, rbuf1, rbuf2, rbuf3,
             wbuf0, wbuf1, sem_fg0, sem_fg1, sem_fw0, sem_fw1,
             sem_g0, sem_g1, sem_g2, sem_g3, sem_w0, sem_w1):
    wid = lax.axis_index("s") * NC + lax.axis_index("c")

    pltpu.sync_copy(keys_hbm, keys_v)

    kk = jnp.int32(K)
    b1, kk = _radix_pass(keys_v, hist_v, 21, 11, 0, 0, kk, True)
    p1 = b1.astype(jnp.uint32)
    b2, kk = _radix_pass(keys_v, hist_v, 10, 11, 21, p1, kk, False)
    p2 = (p1 << 11) | b2.astype(jnp.uint32)
    b3, kk = _radix_pass(keys_v, hist_v, 0, 10, 10, p2, kk, False)
    thresh = (p2 << 10) | b3.astype(jnp.uint32)
    need = kk

    iota = lax.iota(jnp.int32, 16)

    def comp_body(i, carry):
        gt_cnt, tie_cnt = carry
        kv = keys_v[pl.ds(i * 16, 16)]
        gt_m = kv > thresh
        eq_m = kv == thresh
        gt_i = gt_m.astype(jnp.int32)
        eq_i = eq_m.astype(jnp.int32)
        gt_c = plsc.cumsum(gt_i)
        eq_c = plsc.cumsum(eq_i)
        gt_excl = gt_cnt + gt_c - gt_i
        tie_excl = tie_cnt + eq_c - eq_i
        sel = jnp.logical_or(gt_m, jnp.logical_and(eq_m, tie_excl < need))
        pos = gt_excl + jnp.minimum(tie_excl, need)
        plsc.store_scatter(idx_v, [pos], i * 16 + iota, mask=sel)
        return gt_cnt + jnp.max(gt_c), tie_cnt + jnp.max(eq_c)

    lax.fori_loop(0, NV, comp_body, (jnp.int32(0), jnp.int32(0)))

    NF = 20
    fbufs = (fbuf0, fbuf1)
    sem_fg = (sem_fg0, sem_fg1)
    sem_fw = (sem_fw0, sem_fw1)

    def f_chunk(t):
        return jnp.minimum(wid + NW * t, FCHUNKS - 1)

    def f_start(t, b):
        pltpu.async_copy(
            feat_hbm.at[idx_v.at[pl.ds(8 * f_chunk(t), 8)]],
            fbufs[b], sem_fg[b])

    for b in range(2):
        f_start(b, b)

    def f_body(u, _):
        for b in range(2):
            t = 2 * u + b
            pltpu.make_async_copy(feat_hbm.at[pl.ds(0, 8)], fbufs[b],
                                  sem_fg[b]).wait()
            pltpu.async_copy(fbufs[b], xp_hbm.at[pl.ds(8 * f_chunk(t), 8)],
                             sem_fw[b])
            pltpu.make_async_copy(fbufs[b], xp_hbm.at[pl.ds(0, 8)],
                                  sem_fw[b]).wait()

            @pl.when(t + 2 < NF)
            def _(b=b, t=t):
                f_start(t + 2, b)

        return 0

    lax.fori_loop(0, NF // 2, f_body, 0)

    NG = 20
    GROUPS = FCHUNKS
    NROWS = 8 * NG

    rbufs = (rbuf0, rbuf1, rbuf2, rbuf3)
    wbufs = (wbuf0, wbuf1)
    sem_g = (sem_g0, sem_g1, sem_g2, sem_g3)
    sem_w = (sem_w0, sem_w1)

    def out_row(t):
        return 8 * jnp.minimum(wid + NW * (t // 8), GROUPS - 1) + t % 8

    def row_src(t):
        row = out_row(t)
        vec = idx_v[pl.ds((row // 16) * 16, 16)]
        v = jnp.max(jnp.where(iota == row % 16, vec, 0))
        return a_hbm.at[v // 8, v % 8]

    def out_dst(t):
        row = out_row(t)
        return ap_hbm.at[row // 8, row % 8]

    for b in range(4):
        pltpu.async_copy(row_src(b), rbufs[b], sem_g[b])

    def a_body(u, _):
        for b in range(4):
            t = 4 * u + b
            w = b % 2
            pltpu.make_async_copy(a_hbm.at[0, 0], rbufs[b], sem_g[b]).wait()

            @pl.when(t >= 2)
            def _(w=w):
                pltpu.make_async_copy(wbufs[w], ap_hbm.at[0, 0],
                                      sem_w[w]).wait()

            @plsc.parallel_loop(0, CCHUNKS, 1, unroll=8)
            def col_body(j, b=b, w=w):
                off = jnp.minimum(j * 16, K - 16)
                col = idx_v[pl.ds(off, 16)]
                wbufs[w][pl.ds(off, 16)] = plsc.load_gather(rbufs[b], [col])
            pltpu.async_copy(wbufs[w], out_dst(t), sem_w[w])

            @pl.when(t + 4 < NROWS)
            def _(b=b, t=t):
                pltpu.async_copy(row_src(t + 4), rbufs[b], sem_g[b])

        return 0

    lax.fori_loop(0, NROWS // 4, a_body, 0)
    for b in range(2):
        pltpu.make_async_copy(wbufs[b], ap_hbm.at[0, 0], sem_w[b]).wait()


_pool = pl.kernel(
    _sc_body,
    out_type=(
        jax.ShapeDtypeStruct((K, D), jnp.float32),
        jax.ShapeDtypeStruct((625, 8, K), jnp.float32),
    ),
    mesh=plsc.VectorSubcoreMesh(core_axis_name="c", subcore_axis_name="s"),
    compiler_params=pltpu.CompilerParams(needs_layout_passes=False),
    scratch_types=[
        pltpu.VMEM((NPAD,), jnp.uint32),
        pltpu.VMEM((K,), jnp.int32),
        pltpu.VMEM((2048,), jnp.int32),
        pltpu.VMEM((8, D), jnp.float32),
        pltpu.VMEM((8, D), jnp.float32),
        pltpu.VMEM((N,), jnp.float32),
        pltpu.VMEM((N,), jnp.float32),
        pltpu.VMEM((N,), jnp.float32),
        pltpu.VMEM((N,), jnp.float32),
        pltpu.VMEM((K,), jnp.float32),
        pltpu.VMEM((K,), jnp.float32),
    ] + [pltpu.SemaphoreType.DMA] * 10,
)


@jax.jit
def kernel(X, A, kernel):
    p = kernel / jnp.sqrt(jnp.sum(jnp.square(kernel)) + 1e-12)
    keys2d, feat = _score(X, p)
    xp, ap = _pool(keys2d[:, 0], feat, A.reshape(1250, 8, N))
    return xp, ap.reshape(K, K)


# speedup vs baseline: 12.4449x; 1.0234x over previous
import functools

import jax
import jax.numpy as jnp
from jax import lax
from jax.experimental import pallas as pl
from jax.experimental.pallas import tpu as pltpu
from jax.experimental.pallas import tpu_sc as plsc

N = 10000
D = 128
K = 5000
NPAD = 10240
NV = NPAD // 16
NC = 2
NW = 32
FCHUNKS = (K + 7) // 8
ACHUNKS = K // 2
CCHUNKS = (K + 15) // 16


def _score_body(x_ref, p_ref, keys_ref):
    x = x_ref[...]
    pn = p_ref[...]
    y = jnp.dot(x, pn, preferred_element_type=jnp.float32)
    u = lax.bitcast_convert_type(y, jnp.uint32)
    key = jnp.where(u >= jnp.uint32(0x80000000), ~u, u | jnp.uint32(0x80000000))
    keys_ref[...] = jnp.concatenate(
        [lax.bitcast_convert_type(key, jnp.int32),
         jnp.zeros((NPAD - N, 1), jnp.int32)], axis=0)


_score = pl.pallas_call(
    _score_body,
    out_shape=jax.ShapeDtypeStruct((NPAD, 1), jnp.int32),
)


def _radix_pass(keys_v, hist_v, shift, nbits, cand_shift, cand_prefix, kk,
                first):
    nb = 1 << nbits
    nb16 = nb // 16
    zeros = jnp.zeros((16,), jnp.int32)
    ones = jnp.ones((16,), jnp.int32)
    fmask = jnp.uint32(nb - 1)

    def zero_body(i, _):
        for q in range(4):
            hist_v[pl.ds((4 * i + q) * 16, 16)] = zeros
        return 0

    lax.fori_loop(0, nb16 // 4, zero_body, 0)

    def build_body(i, _):
        for q in range(4):
            kv = plsc.bitcast(keys_v[pl.ds((4 * i + q) * 16, 16)],
                              jnp.uint32)
            b = ((kv >> jnp.uint32(shift)) & fmask).astype(jnp.int32)
            if first:
                plsc.addupdate_scatter(hist_v, [b], ones)
            else:
                m = (kv >> jnp.uint32(cand_shift)) == cand_prefix
                plsc.addupdate_scatter(hist_v, [b], ones, mask=m)
        return 0

    lax.fori_loop(0, NV // 4, build_body, 0)

    def scan_body(i, carry):
        for q in range(4):
            run, found, bucket, c_above = carry
            v = nb16 - 1 - (4 * i + q)
            h = hist_v[pl.ds(v * 16, 16)]
            hr = lax.rev(h, (0,))
            s = run + plsc.cumsum(hr)
            okm = s >= kk
            anyok = jnp.max(plsc.all_reduce_population_count(okm)) > 0
            j = jnp.max(plsc.all_reduce_ffs(okm))
            ca = jnp.maximum(run, jnp.max(jnp.where(okm, 0, s)))
            newf = jnp.logical_and(anyok, jnp.logical_not(found))
            bucket = jnp.where(newf, v * 16 + (15 - j), bucket)
            c_above = jnp.where(newf, ca, c_above)
            run = jnp.max(s)
            found = jnp.logical_or(found, anyok)
            carry = (run, found, bucket, c_above)
        return carry

    _, _, bucket, c_above = lax.fori_loop(
        0, nb16 // 4, scan_body,
        (jnp.int32(0), False, jnp.int32(0), jnp.int32(0)))
    return bucket, kk - c_above


def _sc_body(keys_hbm, x_hbm, a_hbm, xp_hbm, ap_hbm,
             keys_v, idx_v, hist_v, fbuf0, fbuf1,
             rbuf0, rbuf1, rbuf2, rbuf3,
             wbuf0, wbuf1, sem_fg0, sem_fg1, sem_fw0, sem_fw1,
             sem_g0, sem_g1, sem_g2, sem_g3, sem_w0, sem_w1):
    wid = lax.axis_index("s") * NC + lax.axis_index("c")

    pltpu.sync_copy(keys_hbm, keys_v)

    kk = jnp.int32(K)
    b1, kk = _radix_pass(keys_v, hist_v, 21, 11, 0, 0, kk, True)
    p1 = b1.astype(jnp.uint32)
    b2, kk = _radix_pass(keys_v, hist_v, 10, 11, 21, p1, kk, False)
    p2 = (p1 << 11) | b2.astype(jnp.uint32)
    b3, kk = _radix_pass(keys_v, hist_v, 0, 10, 10, p2, kk, False)
    thresh = (p2 << 10) | b3.astype(jnp.uint32)
    need = kk

    iota = lax.iota(jnp.int32, 16)

    idx_v[pl.ds(K - 8, 16)] = jnp.zeros((16,), jnp.int32)

    def comp_body(i, carry):
        for q in range(4):
            gt_cnt, tie_cnt = carry
            v = 4 * i + q
            kv = plsc.bitcast(keys_v[pl.ds(v * 16, 16)], jnp.uint32)
            gt_m = kv > thresh
            eq_m = kv == thresh
            gt_i = gt_m.astype(jnp.int32)
            eq_i = eq_m.astype(jnp.int32)
            gt_c = plsc.cumsum(gt_i)
            eq_c = plsc.cumsum(eq_i)
            gt_excl = gt_cnt + gt_c - gt_i
            tie_excl = tie_cnt + eq_c - eq_i
            sel = jnp.logical_or(gt_m, jnp.logical_and(eq_m, tie_excl < need))
            pos = gt_excl + jnp.minimum(tie_excl, need)
            plsc.store_scatter(idx_v, [pos], v * 16 + iota, mask=sel)
            carry = (gt_cnt + jnp.max(gt_c), tie_cnt + jnp.max(eq_c))
        return carry

    lax.fori_loop(0, NV // 4, comp_body, (jnp.int32(0), jnp.int32(0)))

    NF = 20
    fbufs = (fbuf0, fbuf1)
    sem_fg = (sem_fg0, sem_fg1)
    sem_fw = (sem_fw0, sem_fw1)

    def f_chunk(t):
        return jnp.minimum(wid + NW * t, FCHUNKS - 1)

    def f_start(t, b):
        pltpu.async_copy(
            x_hbm.at[idx_v.at[pl.ds(8 * f_chunk(t), 8)]],
            fbufs[b], sem_fg[b])

    for b in range(2):
        f_start(b, b)

    def f_body(u, _):
        for b in range(2):
            t = 2 * u + b
            pltpu.make_async_copy(x_hbm.at[pl.ds(0, 8)], fbufs[b],
                                  sem_fg[b]).wait()
            kg = plsc.bitcast(plsc.load_gather(keys_v, [
                plsc.load_gather(idx_v, [8 * f_chunk(t) + iota])]),
                jnp.uint32)
            ug = jnp.where(kg >= jnp.uint32(0x80000000),
                           kg & jnp.uint32(0x7FFFFFFF), ~kg)
            yg = plsc.bitcast(ug, jnp.float32)
            gv = 1.0 - 2.0 / (jnp.exp(2.0 * yg) + 1.0)
            for r in range(8):
                g = jnp.max(jnp.where(iota == r, gv, -2.0))
                for q in range(D // 16):
                    sl = pl.ds(q * 16, 16)
                    fbufs[b][r, sl] = fbufs[b][r, sl] * g
            pltpu.async_copy(fbufs[b], xp_hbm.at[pl.ds(8 * f_chunk(t), 8)],
                             sem_fw[b])
            pltpu.make_async_copy(fbufs[b], xp_hbm.at[pl.ds(0, 8)],
                                  sem_fw[b]).wait()

            @pl.when(t + 2 < NF)
            def _(b=b, t=t):
                f_start(t + 2, b)

        return 0

    lax.fori_loop(0, NF // 2, f_body, 0)

    NG = 20
    GROUPS = FCHUNKS
    NROWS = 8 * NG

    rbufs = (rbuf0, rbuf1, rbuf2, rbuf3)
    wbufs = (wbuf0, wbuf1)
    sem_g = (sem_g0, sem_g1, sem_g2, sem_g3)
    sem_w = (sem_w0, sem_w1)

    def out_row(t):
        return 8 * jnp.minimum(wid + NW * (t // 8), GROUPS - 1) + t % 8

    def row_src(t):
        row = out_row(t)
        vec = idx_v[pl.ds((row // 16) * 16, 16)]
        v = jnp.max(jnp.where(iota == row % 16, vec, 0))
        return a_hbm.at[v // 8, v % 8]

    def out_dst(t):
        row = out_row(t)
        return ap_hbm.at[row // 8, row % 8]

    for b in range(4):
        pltpu.async_copy(row_src(b), rbufs[b], sem_g[b])

    def a_body(u, _):
        for b in range(4):
            t = 4 * u + b
            w = b % 2
            pltpu.make_async_copy(a_hbm.at[0, 0], rbufs[b], sem_g[b]).wait()

            @pl.when(t >= 2)
            def _(w=w):
                pltpu.make_async_copy(wbufs[w], ap_hbm.at[0, 0],
                                      sem_w[w]).wait()

            @plsc.parallel_loop(0, CCHUNKS, 1, unroll=8)
            def col_body(j, b=b, w=w):
                off = jnp.minimum(j * 16, K - 16)
                col = idx_v[pl.ds(off, 16)]
                wbufs[w][pl.ds(off, 16)] = plsc.load_gather(rbufs[b], [col])
            pltpu.async_copy(wbufs[w], out_dst(t), sem_w[w])

            @pl.when(t + 4 < NROWS)
            def _(b=b, t=t):
                pltpu.async_copy(row_src(t + 4), rbufs[b], sem_g[b])

        return 0

    lax.fori_loop(0, NROWS // 4, a_body, 0)
    for b in range(2):
        pltpu.make_async_copy(wbufs[b], ap_hbm.at[0, 0], sem_w[b]).wait()


_pool = pl.kernel(
    _sc_body,
    out_type=(
        jax.ShapeDtypeStruct((K, D), jnp.float32),
        jax.ShapeDtypeStruct((625, 8, K), jnp.float32),
    ),
    mesh=plsc.VectorSubcoreMesh(core_axis_name="c", subcore_axis_name="s"),
    compiler_params=pltpu.CompilerParams(needs_layout_passes=False),
    scratch_types=[
        pltpu.VMEM((NPAD,), jnp.int32),
        pltpu.VMEM((K + 8,), jnp.int32),
        pltpu.VMEM((2048,), jnp.int32),
        pltpu.VMEM((8, D), jnp.float32),
        pltpu.VMEM((8, D), jnp.float32),
        pltpu.VMEM((N,), jnp.float32),
        pltpu.VMEM((N,), jnp.float32),
        pltpu.VMEM((N,), jnp.float32),
        pltpu.VMEM((N,), jnp.float32),
        pltpu.VMEM((K,), jnp.float32),
        pltpu.VMEM((K,), jnp.float32),
    ] + [pltpu.SemaphoreType.DMA] * 10,
)


@jax.jit
def kernel(X, A, kernel):
    p = kernel / jnp.sqrt(jnp.sum(jnp.square(kernel)) + 1e-12)
    keys2d = _score(X, p)
    xp, ap = _pool(keys2d[:, 0], X, A.reshape(1250, 8, N))
    return xp, ap.reshape(K, K)
